# baseline (speedup 1.0000x reference)
import dataclasses
import jax
import jax.numpy as jnp
from jax.experimental import pallas as pl
from jax.experimental.pallas import tpu as pltpu
from jax.experimental.pallas import tpu_sc as plsc

_GRID = 64
_SCALING = 0.5
_EPS = 1e-10
_B = 1024
_H = 129
_N = 16641
_C = 387
_K = _N // _C

_CONST_CACHE = []


def _build_consts():
    key = jax.random.key(42)
    k1, k2 = jax.random.split(key)
    g = _GRID
    x = jnp.arange(0, g * 2 + 1)
    X = jnp.repeat(x[:, None], g * 2 + 1, axis=1)
    x1 = X - g
    x2 = x1.T
    gm = jnp.concatenate((x2[:, :, None], x1[:, :, None]), axis=2)
    gm = gm.reshape(1, -1, 2).astype(jnp.float32)
    gm = jnp.tile(gm, (_B, 1, 1))
    gm = gm + jax.random.uniform(k1, gm.shape, dtype=jnp.float32)
    u = jax.random.uniform(k2, (_B, _N), dtype=jnp.float32)
    gnoise = -jnp.log(_EPS - jnp.log(u + _EPS))
    gn3 = gnoise.T.reshape(_N, 8, 128)
    t0 = (gm[:, :, 0] * _SCALING).reshape(_B * _N // 128, 128)
    t1 = (gm[:, :, 1] * _SCALING).reshape(_B * _N // 128, 128)
    io32 = jnp.arange(32, dtype=jnp.int32)
    return gn3, t0, t1, io32


def _consts():
    if not _CONST_CACHE:
        try:
            with jax.ensure_compile_time_eval():
                _CONST_CACHE.append(jax.tree.map(jax.block_until_ready,
                                                 _build_consts()))
        except Exception:
            return _build_consts()
    return _CONST_CACHE[0]


def _stats_body(a_ref, gn_ref, m1_ref, s1_ref, m2_ref, s2_ref,
                mi_ref, pr_ref, pw_ref):
    k = pl.program_id(0)

    @pl.when(k == 0)
    def _():
        m1_ref[...] = jnp.full((8, 128), -jnp.inf, jnp.float32)
        s1_ref[...] = jnp.zeros((8, 128), jnp.float32)
        m2_ref[...] = jnp.full((8, 128), -jnp.inf, jnp.float32)
        s2_ref[...] = jnp.zeros((8, 128), jnp.float32)
        mi_ref[...] = jnp.full((8, 128), _N, jnp.int32)

    a = a_ref[...]
    gl = a + gn_ref[...]

    cm1 = jnp.max(gl, axis=0)
    m1o = m1_ref[...]
    m1n = jnp.maximum(m1o, cm1)
    s1_ref[...] = (s1_ref[...] * jnp.exp(m1o - m1n)
                   + jnp.sum(jnp.exp(gl - m1n[None]), axis=0))
    m1_ref[...] = m1n

    cm2 = jnp.max(a, axis=0)
    m2o = m2_ref[...]
    m2n = jnp.maximum(m2o, cm2)
    s2_ref[...] = (s2_ref[...] * jnp.exp(m2o - m2n)
                   + jnp.sum(jnp.exp(a - m2n[None]), axis=0))
    m2_ref[...] = m2n

    col = k * _C + jax.lax.broadcasted_iota(jnp.int32, gl.shape, 0)
    ci = jnp.min(jnp.where(gl == cm1[None], col, _N), axis=0)
    mi = jnp.where(cm1 > m1o, ci, mi_ref[...])
    mi_ref[...] = mi

    bmat = (jax.lax.broadcasted_iota(jnp.int32, (8, 128), 0) * 128
            + jax.lax.broadcasted_iota(jnp.int32, (8, 128), 1))
    p = bmat * _N + mi
    pr_ref[...] = jax.lax.shift_right_logical(p, 7)
    pw_ref[...] = jnp.bitwise_and(p, 127)


def _write_body(a_ref, gn_ref, m1_ref, s1_ref, m2_ref, s2_ref, mi_ref,
                sg_ref, s_ref, oh_ref):
    k = pl.program_id(0)
    a = a_ref[...]
    gl = a + gn_ref[...]

    r1 = 1.0 / s1_ref[...]
    e1 = jnp.exp(gl - m1_ref[...][None])
    sg_ref[...] = jnp.maximum(e1 * r1[None], _EPS)

    r2 = 1.0 / s2_ref[...]
    s_ref[...] = jnp.exp(a - m2_ref[...][None]) * r2[None]

    smax = jnp.maximum(r1, _EPS)
    yval = (1.0 - smax) + smax
    col = k * _C + jax.lax.broadcasted_iota(jnp.int32, gl.shape, 0)
    hot = col == mi_ref[...][None]
    oh_ref[...] = jnp.where(hot, yval[None], 0.0)


def _sc_body(t0_hbm, t1_hbm, pr_hbm, pw_hbm, io_hbm, o0_hbm, o1_hbm,
             pr_v, pw_v, io_v, rows0, rows1, o0_v, o1_v, sem):
    wid = jax.lax.axis_index("s") * 2 + jax.lax.axis_index("c")
    base = wid * 32
    pltpu.sync_copy(pr_hbm.at[pl.ds(base, 32)], pr_v)
    pltpu.sync_copy(pw_hbm.at[pl.ds(base, 32)], pw_v)
    pltpu.sync_copy(io_hbm.at[pl.ds(0, 32)], io_v)
    pltpu.async_copy(t0_hbm.at[pr_v], rows0, sem).wait()
    pltpu.async_copy(t1_hbm.at[pr_v], rows1, sem).wait()
    for c in (0, 16):
        rid = io_v.at[pl.ds(c, 16)][...]
        lid = pw_v.at[pl.ds(c, 16)][...]
        o0_v.at[pl.ds(c, 16)][...] = plsc.load_gather(rows0, [rid, lid])
        o1_v.at[pl.ds(c, 16)][...] = plsc.load_gather(rows1, [rid, lid])
    pltpu.sync_copy(o0_v, o0_hbm.at[pl.ds(base, 32)])
    pltpu.sync_copy(o1_v, o1_hbm.at[pl.ds(base, 32)])


def _sc_gather(t0, t1, prow, plane, io32):
    mesh = plsc.VectorSubcoreMesh(core_axis_name="c", subcore_axis_name="s")
    cp = pltpu.CompilerParams()
    if "needs_layout_passes" in getattr(
            pltpu.CompilerParams, "__dataclass_fields__", {}):
        cp = dataclasses.replace(cp, needs_layout_passes=False)
    f32 = jnp.float32
    fn = pl.kernel(
        _sc_body,
        mesh=mesh,
        out_type=[jax.ShapeDtypeStruct((_B,), f32),
                  jax.ShapeDtypeStruct((_B,), f32)],
        scratch_types=[pltpu.VMEM((32,), jnp.int32),
                       pltpu.VMEM((32,), jnp.int32),
                       pltpu.VMEM((32,), jnp.int32),
                       pltpu.VMEM((32, 128), f32),
                       pltpu.VMEM((32, 128), f32),
                       pltpu.VMEM((32,), f32),
                       pltpu.VMEM((32,), f32),
                       pltpu.SemaphoreType.DMA],
        compiler_params=cp,
    )
    return fn(t0, t1, prow, plane, io32)


def kernel(cnn_out):
    b, c, hh, w = cnn_out.shape
    gn3, t0, t1, io32 = _consts()

    a2 = cnn_out.reshape(b, _N)
    a3 = a2.T.reshape(_N, 8, 128)

    chunk = pl.BlockSpec((_C, 8, 128), lambda i: (i, 0, 0))
    small = pl.BlockSpec((8, 128), lambda i: (0, 0))
    sf32 = jax.ShapeDtypeStruct((8, 128), jnp.float32)
    si32 = jax.ShapeDtypeStruct((8, 128), jnp.int32)

    m1, s1, m2, s2, mi, pr, pw = pl.pallas_call(
        _stats_body,
        grid=(_K,),
        in_specs=[chunk, chunk],
        out_specs=[small] * 7,
        out_shape=[sf32, sf32, sf32, sf32, si32, si32, si32],
        compiler_params=pltpu.CompilerParams(
            dimension_semantics=("arbitrary",)),
    )(a3, gn3)

    big3 = jax.ShapeDtypeStruct((_N, 8, 128), jnp.float32)
    sg3, s3, oh3 = pl.pallas_call(
        _write_body,
        grid=(_K,),
        in_specs=[chunk, chunk] + [small] * 5,
        out_specs=[chunk] * 3,
        out_shape=[big3, big3, big3],
        compiler_params=pltpu.CompilerParams(
            dimension_semantics=("arbitrary",)),
    )(a3, gn3, m1, s1, m2, s2, mi)

    o0, o1 = _sc_gather(t0, t1, pr.reshape(_B), pw.reshape(_B), io32)
    fp = jnp.stack([o0, o1], axis=-1)[None]

    def unt(x3):
        return x3.reshape(_N, _B).T.reshape(b, c, hh, w)

    return (fp, unt(oh3), unt(sg3), unt(s3), a2)


# speedup vs baseline: 1.2901x; 1.2901x over previous
import dataclasses
import jax
import jax.numpy as jnp
from jax.experimental import pallas as pl
from jax.experimental.pallas import tpu as pltpu
from jax.experimental.pallas import tpu_sc as plsc

_GRID = 64
_SCALING = 0.5
_EPS = 1e-10
_B = 1024
_H = 129
_N = 16641
_C = 344
_K = -(-_N // _C)

_CONST_CACHE = []


def _build_consts():
    key = jax.random.key(42)
    k1, k2 = jax.random.split(key)
    g = _GRID
    x = jnp.arange(0, g * 2 + 1)
    X = jnp.repeat(x[:, None], g * 2 + 1, axis=1)
    x1 = X - g
    x2 = x1.T
    gm = jnp.concatenate((x2[:, :, None], x1[:, :, None]), axis=2)
    gm = gm.reshape(1, -1, 2).astype(jnp.float32)
    gm = jnp.tile(gm, (_B, 1, 1))
    gm = gm + jax.random.uniform(k1, gm.shape, dtype=jnp.float32)
    u = jax.random.uniform(k2, (_B, _N), dtype=jnp.float32)
    gnoise = -jnp.log(_EPS - jnp.log(u + _EPS))
    gnt = gnoise.T
    t0 = (gm[:, :, 0] * _SCALING).reshape(_B * _N // 128, 128)
    t1 = (gm[:, :, 1] * _SCALING).reshape(_B * _N // 128, 128)
    io32 = jnp.arange(32, dtype=jnp.int32)
    return gnt, t0, t1, io32


def _consts():
    if not _CONST_CACHE:
        try:
            with jax.ensure_compile_time_eval():
                _CONST_CACHE.append(jax.tree.map(jax.block_until_ready,
                                                 _build_consts()))
        except Exception:
            return _build_consts()
    return _CONST_CACHE[0]


def _stats_body(a_ref, gn_ref, m1_ref, s1_ref, m2_ref, s2_ref,
                mi_ref, pr_ref, pw_ref):
    k = pl.program_id(0)

    @pl.when(k == 0)
    def _():
        m1_ref[...] = jnp.full((1, _B), -jnp.inf, jnp.float32)
        s1_ref[...] = jnp.zeros((1, _B), jnp.float32)
        m2_ref[...] = jnp.full((1, _B), -jnp.inf, jnp.float32)
        s2_ref[...] = jnp.zeros((1, _B), jnp.float32)
        mi_ref[...] = jnp.full((1, _B), _N, jnp.int32)

    col = k * _C + jax.lax.broadcasted_iota(jnp.int32, a_ref.shape, 0)
    valid = col < _N
    a = jnp.where(valid, a_ref[...], -jnp.inf)
    gl = jnp.where(valid, a + gn_ref[...], -jnp.inf)

    cm1 = jnp.max(gl, axis=0, keepdims=True)
    m1o = m1_ref[...]
    m1n = jnp.maximum(m1o, cm1)
    s1_ref[...] = (s1_ref[...] * jnp.exp(m1o - m1n)
                   + jnp.sum(jnp.exp(gl - m1n), axis=0, keepdims=True))
    m1_ref[...] = m1n

    cm2 = jnp.max(a, axis=0, keepdims=True)
    m2o = m2_ref[...]
    m2n = jnp.maximum(m2o, cm2)
    s2_ref[...] = (s2_ref[...] * jnp.exp(m2o - m2n)
                   + jnp.sum(jnp.exp(a - m2n), axis=0, keepdims=True))
    m2_ref[...] = m2n

    ci = jnp.min(jnp.where(gl == cm1, col, _N), axis=0, keepdims=True)
    mi = jnp.where(cm1 > m1o, ci, mi_ref[...])
    mi_ref[...] = mi

    bvec = jax.lax.broadcasted_iota(jnp.int32, (1, _B), 1)
    p = bvec * _N + mi
    pr_ref[...] = jax.lax.shift_right_logical(p, 7)
    pw_ref[...] = jnp.bitwise_and(p, 127)


def _write_body(a_ref, gn_ref, m1_ref, s1_ref, m2_ref, s2_ref, mi_ref,
                sg_ref, s_ref, oh_ref):
    k = pl.program_id(0)
    a = a_ref[...]
    gl = a + gn_ref[...]

    r1 = 1.0 / s1_ref[...]
    e1 = jnp.exp(gl - m1_ref[...])
    sg_ref[...] = jnp.maximum(e1 * r1, _EPS)

    r2 = 1.0 / s2_ref[...]
    s_ref[...] = jnp.exp(a - m2_ref[...]) * r2

    smax = jnp.maximum(r1, _EPS)
    yval = (1.0 - smax) + smax
    col = k * _C + jax.lax.broadcasted_iota(jnp.int32, gl.shape, 0)
    hot = col == mi_ref[...]
    oh_ref[...] = jnp.where(hot, yval, 0.0)


def _sc_body(t0_hbm, t1_hbm, pr_hbm, pw_hbm, io_hbm, o0_hbm, o1_hbm,
             pr_v, pw_v, io_v, rows0, rows1, o0_v, o1_v, sem):
    wid = jax.lax.axis_index("s") * 2 + jax.lax.axis_index("c")
    base = wid * 32
    pltpu.sync_copy(pr_hbm.at[pl.ds(base, 32)], pr_v)
    pltpu.sync_copy(pw_hbm.at[pl.ds(base, 32)], pw_v)
    pltpu.sync_copy(io_hbm.at[pl.ds(0, 32)], io_v)
    pltpu.async_copy(t0_hbm.at[pr_v], rows0, sem).wait()
    pltpu.async_copy(t1_hbm.at[pr_v], rows1, sem).wait()
    for c in (0, 16):
        rid = io_v.at[pl.ds(c, 16)][...]
        lid = pw_v.at[pl.ds(c, 16)][...]
        o0_v.at[pl.ds(c, 16)][...] = plsc.load_gather(rows0, [rid, lid])
        o1_v.at[pl.ds(c, 16)][...] = plsc.load_gather(rows1, [rid, lid])
    pltpu.sync_copy(o0_v, o0_hbm.at[pl.ds(base, 32)])
    pltpu.sync_copy(o1_v, o1_hbm.at[pl.ds(base, 32)])


def _sc_gather(t0, t1, prow, plane, io32):
    mesh = plsc.VectorSubcoreMesh(core_axis_name="c", subcore_axis_name="s")
    cp = pltpu.CompilerParams()
    if "needs_layout_passes" in getattr(
            pltpu.CompilerParams, "__dataclass_fields__", {}):
        cp = dataclasses.replace(cp, needs_layout_passes=False)
    f32 = jnp.float32
    fn = pl.kernel(
        _sc_body,
        mesh=mesh,
        out_type=[jax.ShapeDtypeStruct((_B,), f32),
                  jax.ShapeDtypeStruct((_B,), f32)],
        scratch_types=[pltpu.VMEM((32,), jnp.int32),
                       pltpu.VMEM((32,), jnp.int32),
                       pltpu.VMEM((32,), jnp.int32),
                       pltpu.VMEM((32, 128), f32),
                       pltpu.VMEM((32, 128), f32),
                       pltpu.VMEM((32,), f32),
                       pltpu.VMEM((32,), f32),
                       pltpu.SemaphoreType.DMA],
        compiler_params=cp,
    )
    return fn(t0, t1, prow, plane, io32)


def kernel(cnn_out):
    b, c, hh, w = cnn_out.shape
    gnt, t0, t1, io32 = _consts()

    a2 = cnn_out.reshape(b, _N)
    at = a2.T

    chunk = pl.BlockSpec((_C, _B), lambda i: (i, 0))
    small = pl.BlockSpec((1, _B), lambda i: (0, 0))
    sf32 = jax.ShapeDtypeStruct((1, _B), jnp.float32)
    si32 = jax.ShapeDtypeStruct((1, _B), jnp.int32)

    m1, s1, m2, s2, mi, pr, pw = pl.pallas_call(
        _stats_body,
        grid=(_K,),
        in_specs=[chunk, chunk],
        out_specs=[small] * 7,
        out_shape=[sf32, sf32, sf32, sf32, si32, si32, si32],
        compiler_params=pltpu.CompilerParams(
            dimension_semantics=("arbitrary",)),
    )(at, gnt)

    bigt = jax.ShapeDtypeStruct((_N, _B), jnp.float32)
    sgt, st, oht = pl.pallas_call(
        _write_body,
        grid=(_K,),
        in_specs=[chunk, chunk] + [small] * 5,
        out_specs=[chunk] * 3,
        out_shape=[bigt, bigt, bigt],
        compiler_params=pltpu.CompilerParams(
            dimension_semantics=("arbitrary",)),
    )(at, gnt, m1, s1, m2, s2, mi)

    o0, o1 = _sc_gather(t0, t1, pr.reshape(_B), pw.reshape(_B), io32)
    fp = jnp.stack([o0, o1], axis=-1)[None]

    def unt(xt):
        return xt.T.reshape(b, c, hh, w)

    return (fp, unt(oht), unt(sgt), unt(st), a2)


# speedup vs baseline: 2.9089x; 2.2549x over previous
import dataclasses
import jax
import jax.numpy as jnp
from jax.experimental import pallas as pl
from jax.experimental.pallas import tpu as pltpu
from jax.experimental.pallas import tpu_sc as plsc

_GRID = 64
_SCALING = 0.5
_EPS = 1e-10
_B = 1024
_H = 129
_N = 16641
_C = 344
_K = -(-_N // _C)

_CONST_CACHE = []


def _build_consts():
    key = jax.random.key(42)
    k1, k2 = jax.random.split(key)
    g = _GRID
    x = jnp.arange(0, g * 2 + 1)
    X = jnp.repeat(x[:, None], g * 2 + 1, axis=1)
    x1 = X - g
    x2 = x1.T
    gm = jnp.concatenate((x2[:, :, None], x1[:, :, None]), axis=2)
    gm = gm.reshape(1, -1, 2).astype(jnp.float32)
    gm = jnp.tile(gm, (_B, 1, 1))
    gm = gm + jax.random.uniform(k1, gm.shape, dtype=jnp.float32)
    u = jax.random.uniform(k2, (_B, _N), dtype=jnp.float32)
    gnoise = -jnp.log(_EPS - jnp.log(u + _EPS))
    gnt = gnoise.T
    t0 = (gm[:, :, 0] * _SCALING).reshape(_B * _N // 128, 128)
    t1 = (gm[:, :, 1] * _SCALING).reshape(_B * _N // 128, 128)
    io32 = jnp.arange(32, dtype=jnp.int32)
    return gnt, t0, t1, io32


def _consts():
    if not _CONST_CACHE:
        try:
            with jax.ensure_compile_time_eval():
                _CONST_CACHE.append(jax.tree.map(jax.block_until_ready,
                                                 _build_consts()))
        except Exception:
            return _build_consts()
    return _CONST_CACHE[0]


def _stats_body(a_ref, gn_ref, m1_ref, s1_ref, m2_ref, s2_ref,
                mi_ref, pr_ref, pw_ref):
    k = pl.program_id(0)

    @pl.when(k == 0)
    def _():
        m1_ref[...] = jnp.full((1, _B), -jnp.inf, jnp.float32)
        s1_ref[...] = jnp.zeros((1, _B), jnp.float32)
        m2_ref[...] = jnp.full((1, _B), -jnp.inf, jnp.float32)
        s2_ref[...] = jnp.zeros((1, _B), jnp.float32)
        mi_ref[...] = jnp.full((1, _B), _N, jnp.int32)

    col = k * _C + jax.lax.broadcasted_iota(jnp.int32, a_ref.shape, 0)
    valid = col < _N
    a = jnp.where(valid, a_ref[...], -jnp.inf)
    gl = jnp.where(valid, a + gn_ref[...], -jnp.inf)

    cm1 = jnp.max(gl, axis=0, keepdims=True)
    m1o = m1_ref[...]
    m1n = jnp.maximum(m1o, cm1)
    s1_ref[...] = (s1_ref[...] * jnp.exp(m1o - m1n)
                   + jnp.sum(jnp.exp(gl - m1n), axis=0, keepdims=True))
    m1_ref[...] = m1n

    cm2 = jnp.max(a, axis=0, keepdims=True)
    m2o = m2_ref[...]
    m2n = jnp.maximum(m2o, cm2)
    s2_ref[...] = (s2_ref[...] * jnp.exp(m2o - m2n)
                   + jnp.sum(jnp.exp(a - m2n), axis=0, keepdims=True))
    m2_ref[...] = m2n

    ci = jnp.min(jnp.where(gl == cm1, col, _N), axis=0, keepdims=True)
    mi = jnp.where(cm1 > m1o, ci, mi_ref[...])
    mi_ref[...] = mi

    bvec = jax.lax.broadcasted_iota(jnp.int32, (1, _B), 1)
    p = bvec * _N + mi
    pr_ref[...] = jax.lax.shift_right_logical(p, 7)
    pw_ref[...] = jnp.bitwise_and(p, 127)


def _write_body(a_ref, gn_ref, m1_ref, s1_ref, m2_ref, s2_ref, mi_ref,
                sg_ref, s_ref, oh_ref):
    k = pl.program_id(0)
    a = a_ref[...]
    gl = a + gn_ref[...]

    r1 = 1.0 / s1_ref[...]
    e1 = jnp.exp(gl - m1_ref[...])
    sg_ref[...] = jnp.maximum(e1 * r1, _EPS).reshape(_C * 8, 128)

    r2 = 1.0 / s2_ref[...]
    s_ref[...] = (jnp.exp(a - m2_ref[...]) * r2).reshape(_C * 8, 128)

    smax = jnp.maximum(r1, _EPS)
    yval = (1.0 - smax) + smax
    col = k * _C + jax.lax.broadcasted_iota(jnp.int32, gl.shape, 0)
    hot = col == mi_ref[...]
    oh_ref[...] = jnp.where(hot, yval, 0.0).reshape(_C * 8, 128)


def _sc_body(t0_hbm, t1_hbm, pr_hbm, pw_hbm, io_hbm, o0_hbm, o1_hbm,
             pr_v, pw_v, io_v, rows0, rows1, o0_v, o1_v, sem):
    wid = jax.lax.axis_index("s") * 2 + jax.lax.axis_index("c")
    base = wid * 32
    pltpu.sync_copy(pr_hbm.at[pl.ds(base, 32)], pr_v)
    pltpu.sync_copy(pw_hbm.at[pl.ds(base, 32)], pw_v)
    pltpu.sync_copy(io_hbm.at[pl.ds(0, 32)], io_v)
    pltpu.async_copy(t0_hbm.at[pr_v], rows0, sem).wait()
    pltpu.async_copy(t1_hbm.at[pr_v], rows1, sem).wait()
    for c in (0, 16):
        rid = io_v.at[pl.ds(c, 16)][...]
        lid = pw_v.at[pl.ds(c, 16)][...]
        o0_v.at[pl.ds(c, 16)][...] = plsc.load_gather(rows0, [rid, lid])
        o1_v.at[pl.ds(c, 16)][...] = plsc.load_gather(rows1, [rid, lid])
    pltpu.sync_copy(o0_v, o0_hbm.at[pl.ds(base, 32)])
    pltpu.sync_copy(o1_v, o1_hbm.at[pl.ds(base, 32)])


def _sc_gather(t0, t1, prow, plane, io32):
    mesh = plsc.VectorSubcoreMesh(core_axis_name="c", subcore_axis_name="s")
    cp = pltpu.CompilerParams()
    if "needs_layout_passes" in getattr(
            pltpu.CompilerParams, "__dataclass_fields__", {}):
        cp = dataclasses.replace(cp, needs_layout_passes=False)
    f32 = jnp.float32
    fn = pl.kernel(
        _sc_body,
        mesh=mesh,
        out_type=[jax.ShapeDtypeStruct((_B,), f32),
                  jax.ShapeDtypeStruct((_B,), f32)],
        scratch_types=[pltpu.VMEM((32,), jnp.int32),
                       pltpu.VMEM((32,), jnp.int32),
                       pltpu.VMEM((32,), jnp.int32),
                       pltpu.VMEM((32, 128), f32),
                       pltpu.VMEM((32, 128), f32),
                       pltpu.VMEM((32,), f32),
                       pltpu.VMEM((32,), f32),
                       pltpu.SemaphoreType.DMA],
        compiler_params=cp,
    )
    return fn(t0, t1, prow, plane, io32)


def kernel(cnn_out):
    b, c, hh, w = cnn_out.shape
    gnt, t0, t1, io32 = _consts()

    a2 = cnn_out.reshape(b, _N)
    at = a2.T

    chunk = pl.BlockSpec((_C, _B), lambda i: (i, 0))
    small = pl.BlockSpec((1, _B), lambda i: (0, 0))
    sf32 = jax.ShapeDtypeStruct((1, _B), jnp.float32)
    si32 = jax.ShapeDtypeStruct((1, _B), jnp.int32)

    m1, s1, m2, s2, mi, pr, pw = pl.pallas_call(
        _stats_body,
        grid=(_K,),
        in_specs=[chunk, chunk],
        out_specs=[small] * 7,
        out_shape=[sf32, sf32, sf32, sf32, si32, si32, si32],
        compiler_params=pltpu.CompilerParams(
            dimension_semantics=("arbitrary",)),
    )(at, gnt)

    bigt = jax.ShapeDtypeStruct((_N * 8, 128), jnp.float32)
    out_chunk = pl.BlockSpec((_C * 8, 128), lambda i: (i, 0))
    sgt, st, oht = pl.pallas_call(
        _write_body,
        grid=(_K,),
        in_specs=[chunk, chunk] + [small] * 5,
        out_specs=[out_chunk] * 3,
        out_shape=[bigt, bigt, bigt],
        compiler_params=pltpu.CompilerParams(
            dimension_semantics=("arbitrary",)),
    )(at, gnt, m1, s1, m2, s2, mi)

    o0, o1 = _sc_gather(t0, t1, pr.reshape(_B), pw.reshape(_B), io32)
    fp = jnp.stack([o0, o1], axis=-1)[None]

    def unt(xt):
        return xt.T.reshape(b, c, hh, w)

    def unt(xv):
        x3 = xv.reshape(_N, 8, 128)
        return jnp.transpose(x3, (1, 2, 0)).reshape(b, c, hh, w)

    return (fp, unt(oht), unt(sgt), unt(st), a2)
